# baseline (speedup 1.0000x reference)
import functools

import jax
import jax.numpy as jnp
from jax import lax
from jax.experimental import pallas as pl
from jax.experimental.pallas import tpu as pltpu
from jax.experimental.pallas import tpu_sc as plsc

N_CAT = 20
N_CONT = 6
VOCAB = 100000
B = 16384
D = 32
H = 20
NF = N_CAT + N_CONT

NC = 2
NS = 16
NW = NC * NS
BPW = B // NW
CH = 128
NCH = BPW // CH


_TB = 2048


def _mlp_body(x_ref, w1_ref, b1_ref, w2_ref, b2_ref, o_ref):
    w1 = w1_ref[...]
    b1 = b1_ref[...]
    b2 = b2_ref[...]
    for i in range(N_CONT):
        x = x_ref[i, :]
        h = jnp.tanh(x[:, None] * w1[i][None, :] + b1[i][None, :])
        o = jax.lax.dot_general(h, w2_ref[i], (((1,), (0,)), ((), ())),
                                preferred_element_type=jnp.float32)
        o_ref[i, :, :] = o + b2[i][None, :]


def _mlp_tc(xs, w1s, b1s, w2s, b2s):
    grid = (B // _TB,)
    return pl.pallas_call(
        _mlp_body,
        grid=grid,
        in_specs=[
            pl.BlockSpec((N_CONT, _TB), lambda ib: (0, ib)),
            pl.BlockSpec((N_CONT, H), lambda ib: (0, 0)),
            pl.BlockSpec((N_CONT, H), lambda ib: (0, 0)),
            pl.BlockSpec((N_CONT, H, D), lambda ib: (0, 0, 0)),
            pl.BlockSpec((N_CONT, D), lambda ib: (0, 0)),
        ],
        out_specs=pl.BlockSpec((N_CONT, _TB, D), lambda ib: (0, ib, 0)),
        out_shape=jax.ShapeDtypeStruct((N_CONT, B, D), jnp.float32),
    )(xs, w1s, b1s, w2s, b2s)


def _sc_body(idx_hbm, *rest):
    tables = rest[:N_CAT]
    cont_hbm = rest[N_CAT]
    out_hbm = rest[N_CAT + 1]
    idxb, rows0, rows1, crow, sem0, sem1 = rest[N_CAT + 2:]

    wid = lax.axis_index("s") * NC + lax.axis_index("c")
    base = wid * BPW

    pltpu.sync_copy(idx_hbm.at[:, wid], idxb)

    rows = (rows0, rows1)
    sems = (sem0, sem1)
    steps = [(f, k) for f in range(N_CAT) for k in range(NCH)]
    copies = []
    for s, (f, k) in enumerate(steps):
        copies.append(
            pltpu.async_copy(tables[f].at[idxb.at[f, k]], rows[s % 2], sems[s % 2])
        )
        if s >= 1:
            pf, pk = steps[s - 1]
            copies[s - 1].wait()
            pltpu.sync_copy(rows[(s - 1) % 2],
                            out_hbm.at[pl.ds(base + pk * CH, CH), pf, :])
    pf, pk = steps[-1]
    copies[-1].wait()
    pltpu.sync_copy(rows[(len(steps) - 1) % 2],
                    out_hbm.at[pl.ds(base + pk * CH, CH), pf, :])

    for i in range(N_CONT):
        pltpu.sync_copy(cont_hbm.at[i, pl.ds(base, BPW), :], crow)
        pltpu.sync_copy(crow, out_hbm.at[pl.ds(base, BPW), N_CAT + i, :])


def _sc_assemble(idx_all, tables, cont_out):
    mesh = plsc.VectorSubcoreMesh(core_axis_name="c", subcore_axis_name="s")
    fn = functools.partial(
        pl.kernel,
        out_type=jax.ShapeDtypeStruct((B, NF, D), jnp.float32),
        mesh=mesh,
        compiler_params=pltpu.CompilerParams(use_tc_tiling_on_sc=False),
        scratch_types=[
            pltpu.VMEM((N_CAT, NCH, CH), jnp.int32),
            pltpu.VMEM((CH, D), jnp.float32),
            pltpu.VMEM((CH, D), jnp.float32),
            pltpu.VMEM((BPW, D), jnp.float32),
            pltpu.SemaphoreType.DMA,
            pltpu.SemaphoreType.DMA,
        ],
    )(_sc_body)
    return fn(idx_all, *tables, cont_out)


def kernel(cat_0, cat_1, cat_2, cat_3, cat_4, cat_5, cat_6, cat_7, cat_8, cat_9, cat_10, cat_11, cat_12, cat_13, cat_14, cat_15, cat_16, cat_17, cat_18, cat_19, W_cat_0, W_cat_1, W_cat_2, W_cat_3, W_cat_4, W_cat_5, W_cat_6, W_cat_7, W_cat_8, W_cat_9, W_cat_10, W_cat_11, W_cat_12, W_cat_13, W_cat_14, W_cat_15, W_cat_16, W_cat_17, W_cat_18, W_cat_19, cont_0, W1_0, b1_0, W2_0, b2_0, cont_1, W1_1, b1_1, W2_1, b2_1, cont_2, W1_2, b1_2, W2_2, b2_2, cont_3, W1_3, b1_3, W2_3, b2_3, cont_4, W1_4, b1_4, W2_4, b2_4, cont_5, W1_5, b1_5, W2_5, b2_5):
    cats = [cat_0, cat_1, cat_2, cat_3, cat_4, cat_5, cat_6, cat_7, cat_8,
            cat_9, cat_10, cat_11, cat_12, cat_13, cat_14, cat_15, cat_16,
            cat_17, cat_18, cat_19]
    tables = [W_cat_0, W_cat_1, W_cat_2, W_cat_3, W_cat_4, W_cat_5, W_cat_6,
              W_cat_7, W_cat_8, W_cat_9, W_cat_10, W_cat_11, W_cat_12,
              W_cat_13, W_cat_14, W_cat_15, W_cat_16, W_cat_17, W_cat_18,
              W_cat_19]
    conts = [cont_0, cont_1, cont_2, cont_3, cont_4, cont_5]
    w1s = jnp.stack([W1_0, W1_1, W1_2, W1_3, W1_4, W1_5]).reshape(N_CONT, H)
    b1s = jnp.stack([b1_0, b1_1, b1_2, b1_3, b1_4, b1_5])
    w2s = jnp.stack([W2_0, W2_1, W2_2, W2_3, W2_4, W2_5])
    b2s = jnp.stack([b2_0, b2_1, b2_2, b2_3, b2_4, b2_5])

    idx_all = jnp.stack(cats).reshape(N_CAT, NW, NCH, CH)
    xs = jnp.stack(conts)

    cont_out = _mlp_tc(xs, w1s, b1s, w2s, b2s)
    return _sc_assemble(idx_all, tables, cont_out)


# speedup vs baseline: 1.6524x; 1.6524x over previous
import functools

import jax
import jax.numpy as jnp
from jax import lax
from jax.experimental import pallas as pl
from jax.experimental.pallas import tpu as pltpu
from jax.experimental.pallas import tpu_sc as plsc

N_CAT = 20
N_CONT = 6
VOCAB = 100000
B = 16384
D = 32
H = 20
NF = N_CAT + N_CONT

NC = 2
NS = 16
NW = NC * NS
CHUNK = 4096
NCHUNK = B // CHUNK


_TB = 2048


def _mlp_body(x_ref, w1_ref, b1_ref, w2t_ref, b2_ref, o_ref):
    w1 = w1_ref[...]
    b1 = b1_ref[...]
    b2 = b2_ref[...]
    for i in range(N_CONT):
        x = x_ref[i, :]
        hT = jnp.tanh(w1[i][:, None] * x[None, :] + b1[i][:, None])
        oT = jax.lax.dot_general(w2t_ref[i], hT, (((1,), (0,)), ((), ())),
                                 preferred_element_type=jnp.float32)
        o_ref[i, :, :] = oT + b2[i][:, None]


def _mlp_tc(xs, w1s, b1s, w2ts, b2s):
    return pl.pallas_call(
        _mlp_body,
        grid=(B // _TB,),
        in_specs=[
            pl.BlockSpec((N_CONT, _TB), lambda ib: (0, ib)),
            pl.BlockSpec((N_CONT, H), lambda ib: (0, 0)),
            pl.BlockSpec((N_CONT, H), lambda ib: (0, 0)),
            pl.BlockSpec((N_CONT, D, H), lambda ib: (0, 0, 0)),
            pl.BlockSpec((N_CONT, D), lambda ib: (0, 0)),
        ],
        out_specs=pl.BlockSpec((N_CONT, D, _TB), lambda ib: (0, 0, ib)),
        out_shape=jax.ShapeDtypeStruct((N_CONT, D, B), jnp.float32),
    )(xs, w1s, b1s, w2ts, b2s)


def _sc_body(*refs):
    idxs = refs[:N_CAT]
    tabs = refs[N_CAT:2 * N_CAT]
    cont_hbm = refs[2 * N_CAT]
    out_hbm = refs[2 * N_CAT + 1]
    lineb, idxb, ob0, ob1, sem0, sem1 = refs[2 * N_CAT + 2:]

    wid = lax.axis_index("s") * NC + lax.axis_index("c")

    obufs = (ob0, ob1)
    sems = (sem0, sem1)
    pending = [None, None]
    zc = jnp.zeros((16,), jnp.int32)

    for f in range(N_CAT):
        pltpu.sync_copy(idxs[f], idxb)
        pltpu.sync_copy(tabs[f].at[pl.ds(wid, 1), :], lineb)
        for c in range(NCHUNK):
            p = c % 2
            if pending[p] is not None:
                pending[p].wait()
                pending[p] = None
            buf = obufs[p]

            @pl.loop(0, CHUNK // 16, unroll=8)
            def _gather(v, c=c, buf=buf):
                iv = idxb[pl.ds(c * CHUNK + v * 16, 16)]
                buf[pl.ds(v * 16, 16)] = plsc.load_gather(lineb, [zc, iv])

            pending[p] = pltpu.async_copy(
                buf, out_hbm.at[f, wid, pl.ds(c * CHUNK, CHUNK)], sems[p])
    for p in range(2):
        if pending[p] is not None:
            pending[p].wait()
            pending[p] = None

    for i in range(N_CONT):
        pltpu.sync_copy(cont_hbm.at[i, wid, :], out_hbm.at[N_CAT + i, wid, :])


def _sc_assemble(cats, tabsT, contT):
    mesh = plsc.VectorSubcoreMesh(core_axis_name="c", subcore_axis_name="s")
    fn = functools.partial(
        pl.kernel,
        out_type=jax.ShapeDtypeStruct((NF, D, B), jnp.float32),
        mesh=mesh,
        compiler_params=pltpu.CompilerParams(use_tc_tiling_on_sc=True,
                                             needs_layout_passes=False),
        scratch_types=[
            pltpu.VMEM((1, VOCAB), jnp.float32),
            pltpu.VMEM((B,), jnp.int32),
            pltpu.VMEM((CHUNK,), jnp.float32),
            pltpu.VMEM((CHUNK,), jnp.float32),
            pltpu.SemaphoreType.DMA,
            pltpu.SemaphoreType.DMA,
        ],
    )(_sc_body)
    return fn(*cats, *tabsT, contT)


def kernel(cat_0, cat_1, cat_2, cat_3, cat_4, cat_5, cat_6, cat_7, cat_8, cat_9, cat_10, cat_11, cat_12, cat_13, cat_14, cat_15, cat_16, cat_17, cat_18, cat_19, W_cat_0, W_cat_1, W_cat_2, W_cat_3, W_cat_4, W_cat_5, W_cat_6, W_cat_7, W_cat_8, W_cat_9, W_cat_10, W_cat_11, W_cat_12, W_cat_13, W_cat_14, W_cat_15, W_cat_16, W_cat_17, W_cat_18, W_cat_19, cont_0, W1_0, b1_0, W2_0, b2_0, cont_1, W1_1, b1_1, W2_1, b2_1, cont_2, W1_2, b1_2, W2_2, b2_2, cont_3, W1_3, b1_3, W2_3, b2_3, cont_4, W1_4, b1_4, W2_4, b2_4, cont_5, W1_5, b1_5, W2_5, b2_5):
    cats = [cat_0, cat_1, cat_2, cat_3, cat_4, cat_5, cat_6, cat_7, cat_8,
            cat_9, cat_10, cat_11, cat_12, cat_13, cat_14, cat_15, cat_16,
            cat_17, cat_18, cat_19]
    tables = [W_cat_0, W_cat_1, W_cat_2, W_cat_3, W_cat_4, W_cat_5, W_cat_6,
              W_cat_7, W_cat_8, W_cat_9, W_cat_10, W_cat_11, W_cat_12,
              W_cat_13, W_cat_14, W_cat_15, W_cat_16, W_cat_17, W_cat_18,
              W_cat_19]
    conts = [cont_0, cont_1, cont_2, cont_3, cont_4, cont_5]
    w1s = jnp.stack([W1_0, W1_1, W1_2, W1_3, W1_4, W1_5]).reshape(N_CONT, H)
    b1s = jnp.stack([b1_0, b1_1, b1_2, b1_3, b1_4, b1_5])
    w2ts = jnp.stack([W2_0.T, W2_1.T, W2_2.T, W2_3.T, W2_4.T, W2_5.T])
    b2s = jnp.stack([b2_0, b2_1, b2_2, b2_3, b2_4, b2_5])
    xs = jnp.stack(conts)

    tabsT = [jnp.transpose(t) for t in tables]

    contT = _mlp_tc(xs, w1s, b1s, w2ts, b2s)
    out_t = _sc_assemble(cats, tabsT, contT)
    return jnp.transpose(out_t, (2, 0, 1))
